# Initial kernel scaffold; baseline (speedup 1.0000x reference)
import jax
import jax.numpy as jnp
from jax.experimental import pallas as pl


def kernel(input, input_mask, embed):
    raise NotImplementedError("write your pallas kernel here")


# speedup vs baseline: 1.3084x; 1.3084x over previous
import functools

import jax
import jax.numpy as jnp
from jax import lax
from jax.experimental import pallas as pl
from jax.experimental.pallas import tpu as pltpu
from jax.experimental.pallas import tpu_sc as plsc


def _tc_body(nblk, total_elems, x_ref, e_ref, m_ref, ind_ref, indg_ref,
             diff_ref, eff_ref, counts_ref, acc_ref):
    i = pl.program_id(0)

    @pl.when(i == 0)
    def _init():
        counts_ref[...] = jnp.zeros_like(counts_ref)
        acc_ref[0] = 0.0
        acc_ref[1] = 0.0

    x = x_ref[...]
    e = e_ref[...]
    xsq = jnp.sum(x * x, axis=1, keepdims=True)
    esq = jnp.sum(e * e, axis=0, keepdims=True)
    xe = jnp.dot(x, e, preferred_element_type=jnp.float32)
    neg = (2.0 * xe - xsq) - esq
    ind = jnp.argmax(neg, axis=1).astype(jnp.int32)
    ind_ref[0, 0, :] = ind

    mrow = m_ref[0, 0, :]
    r_, ne_ = neg.shape
    indg_ref[0, 0, :] = jnp.where(mrow > 0.0, ind, jnp.int32(ne_))

    min_dist = -jnp.max(neg, axis=1)
    onehot = (lax.broadcasted_iota(jnp.int32, (r_, ne_), 1)
              == ind[:, None]).astype(jnp.float32)
    counts_ref[0, :] += jnp.sum(onehot * mrow[:, None], axis=0)
    acc_ref[0] += jnp.sum(mrow * min_dist)
    acc_ref[1] += jnp.sum(mrow)

    @pl.when(i == nblk - 1)
    def _fin():
        diff_ref[...] = jnp.full((1, 1), acc_ref[0] / total_elems,
                                 jnp.float32)
        mcount = jnp.maximum(acc_ref[1], 1.0)
        mu = counts_ref[0, :] / mcount
        eff_ref[...] = jnp.full((1, 1), 1.0, jnp.float32) / jnp.sum(mu * mu)


def _tc_stage(flatten, embed, mask3, block_rows):
    n, d = flatten.shape
    ne = embed.shape[1]
    nblk = n // block_rows
    total = float(n * d)
    return pl.pallas_call(
        functools.partial(_tc_body, nblk, total),
        grid=(nblk,),
        in_specs=[
            pl.BlockSpec((block_rows, d), lambda i: (i, 0)),
            pl.BlockSpec((d, ne), lambda i: (0, 0)),
            pl.BlockSpec((1, 1, block_rows), lambda i: (i, 0, 0)),
        ],
        out_specs=[
            pl.BlockSpec((1, 1, block_rows), lambda i: (i, 0, 0)),
            pl.BlockSpec((1, 1, block_rows), lambda i: (i, 0, 0)),
            pl.BlockSpec((1, 1), lambda i: (0, 0)),
            pl.BlockSpec((1, 1), lambda i: (0, 0)),
        ],
        out_shape=[
            jax.ShapeDtypeStruct((nblk, 1, block_rows), jnp.int32),
            jax.ShapeDtypeStruct((nblk, 1, block_rows), jnp.int32),
            jax.ShapeDtypeStruct((1, 1), jnp.float32),
            jax.ShapeDtypeStruct((1, 1), jnp.float32),
        ],
        scratch_shapes=[
            pltpu.VMEM((1, ne), jnp.float32),
            pltpu.SMEM((2,), jnp.float32),
        ],
    )(flatten, embed, mask3)


_CHUNK = 128


def _make_sc_gather(n, d):
    info = plsc.get_sparse_core_info()
    nw = info.num_cores * info.num_subcores
    b_per_w = n // nw
    n_chunks = b_per_w // _CHUNK
    mesh = plsc.VectorSubcoreMesh(core_axis_name="c", subcore_axis_name="s")

    @functools.partial(
        pl.kernel,
        mesh=mesh,
        out_type=jax.ShapeDtypeStruct((n, d), jnp.float32),
        scratch_types=[
            pltpu.VMEM((_CHUNK,), jnp.int32),
            pltpu.VMEM((_CHUNK, d), jnp.float32),
            pltpu.SemaphoreType.DMA,
        ],
    )
    def sc_gather(emb_hbm, ind_hbm, out_hbm, idx_v, q_v, sem):
        wid = lax.axis_index("s") * info.num_cores + lax.axis_index("c")
        base = wid * b_per_w
        for ch in range(n_chunks):
            off = base + ch * _CHUNK
            pltpu.sync_copy(ind_hbm.at[pl.ds(off, _CHUNK)], idx_v)
            pltpu.async_copy(emb_hbm.at[idx_v], q_v, sem).wait()
            pltpu.sync_copy(q_v, out_hbm.at[pl.ds(off, _CHUNK)])

    return sc_gather


def kernel(input, input_mask, embed):
    t, b, d = input.shape
    ne = embed.shape[1]
    n = t * b
    block_rows = 512

    flatten = input.reshape(n, d)
    mask_flat = input_mask.reshape(n).astype(jnp.float32)
    mask3 = mask_flat.reshape(n // block_rows, 1, block_rows)

    ind3, indg3, diff, eff = _tc_stage(flatten, embed, mask3, block_rows)
    ind = ind3.reshape(n)
    ind_gather = indg3.reshape(n)

    table = jnp.concatenate(
        [embed.T, jnp.zeros((8, d), jnp.float32)], axis=0)
    q_st = _make_sc_gather(n, d)(table, ind_gather)

    return (q_st.reshape(t, b, d), diff[0, 0], ind, eff[0, 0])
